# baseline (speedup 1.0000x reference)
import functools

import jax
import jax.numpy as jnp
from jax import lax
from jax.experimental import pallas as pl
from jax.experimental.pallas import tpu as pltpu
from jax.experimental.pallas import tpu_sc as plsc

_B = 4096
_I = 200
_H = 64


def _make_sc_gather(D, B):
    info = plsc.get_sparse_core_info()
    NC, NS = info.num_cores, info.num_subcores
    NW = NC * NS
    b_per_w = B // NW
    mesh = plsc.VectorSubcoreMesh(core_axis_name="c", subcore_axis_name="s")

    @functools.partial(
        pl.kernel,
        mesh=mesh,
        compiler_params=pltpu.CompilerParams(use_tc_tiling_on_sc=False),
        out_type=jax.ShapeDtypeStruct((B, D), jnp.float32),
        scratch_types=[
            pltpu.VMEM((b_per_w,), jnp.int32),
            pltpu.VMEM((b_per_w, D), jnp.float32),
            pltpu.SemaphoreType.DMA,
        ],
    )
    def gather_kernel(idx_hbm, table_hbm, out_hbm, idx_v, rows_v, sem):
        wid = lax.axis_index("s") * NC + lax.axis_index("c")
        base = wid * b_per_w
        pltpu.sync_copy(idx_hbm.at[pl.ds(base, b_per_w)], idx_v)
        pltpu.async_copy(table_hbm.at[idx_v], rows_v, sem).wait()
        pltpu.sync_copy(rows_v, out_hbm.at[pl.ds(base, b_per_w)])

    return gather_kernel


def _bmm_body(iemb_ref, uemb_ref, out_ref):
    out_ref[...] = jnp.sum(iemb_ref[...] * uemb_ref[...][:, None, :], axis=-1)


def _tc_bmm(iemb, uemb, block_b=256):
    B, I, H = iemb.shape
    grid = (B // block_b,)
    return pl.pallas_call(
        _bmm_body,
        grid=grid,
        in_specs=[
            pl.BlockSpec((block_b, I, H), lambda i: (i, 0, 0)),
            pl.BlockSpec((block_b, H), lambda i: (i, 0)),
        ],
        out_specs=pl.BlockSpec((block_b, I), lambda i: (i, 0)),
        out_shape=jax.ShapeDtypeStruct((B, I), jnp.float32),
    )(iemb, uemb)


def kernel(userid_input, iemb, uembedding_weight):
    idx = userid_input.reshape(-1)
    gather = _make_sc_gather(_H, _B)
    uemb = gather(idx, uembedding_weight)
    return _tc_bmm(iemb, uemb)
